# baseline (speedup 1.0000x reference)
import jax
import jax.numpy as jnp
from jax.experimental import pallas as pl


def _pad_body(in_ref, out_ref):
    x = in_ref[0]
    xr = jnp.concatenate([x[-1:, :], x, x[:1, :]], axis=0)
    out_ref[0] = jnp.concatenate([xr[:, -1:], xr, xr[:, :1]], axis=1)


def kernel(inputs):
    b, h, w = inputs.shape
    return pl.pallas_call(
        _pad_body,
        grid=(b,),
        in_specs=[pl.BlockSpec((1, h, w), lambda i: (i, 0, 0))],
        out_specs=pl.BlockSpec((1, h + 2, w + 2), lambda i: (i, 0, 0)),
        out_shape=jax.ShapeDtypeStruct((b, h + 2, w + 2), inputs.dtype),
    )(inputs)


# speedup vs baseline: 1.2034x; 1.2034x over previous
import jax
import jax.numpy as jnp
from jax.experimental import pallas as pl


_BS = 4


def _pad_body(in_ref, out_ref):
    x = in_ref[...]
    xr = jnp.concatenate([x[:, -1:, :], x, x[:, :1, :]], axis=1)
    out_ref[...] = jnp.concatenate([xr[:, :, -1:], xr, xr[:, :, :1]], axis=2)


def kernel(inputs):
    b, h, w = inputs.shape
    return pl.pallas_call(
        _pad_body,
        grid=(b // _BS,),
        in_specs=[pl.BlockSpec((_BS, h, w), lambda i: (i, 0, 0))],
        out_specs=pl.BlockSpec((_BS, h + 2, w + 2), lambda i: (i, 0, 0)),
        out_shape=jax.ShapeDtypeStruct((b, h + 2, w + 2), inputs.dtype),
    )(inputs)


# speedup vs baseline: 1.2162x; 1.0106x over previous
import jax
import jax.numpy as jnp
from jax.experimental import pallas as pl


_BS = 8


def _pad_body(in_ref, out_ref):
    x = in_ref[...]
    xr = jnp.concatenate([x[:, -1:, :], x, x[:, :1, :]], axis=1)
    out_ref[...] = jnp.concatenate([xr[:, :, -1:], xr, xr[:, :, :1]], axis=2)


def kernel(inputs):
    b, h, w = inputs.shape
    return pl.pallas_call(
        _pad_body,
        grid=(b // _BS,),
        in_specs=[pl.BlockSpec((_BS, h, w), lambda i: (i, 0, 0))],
        out_specs=pl.BlockSpec((_BS, h + 2, w + 2), lambda i: (i, 0, 0)),
        out_shape=jax.ShapeDtypeStruct((b, h + 2, w + 2), inputs.dtype),
    )(inputs)
